# baseline (speedup 1.0000x reference)
import jax
import jax.numpy as jnp
from jax import lax
from jax.experimental import pallas as pl
from jax.experimental.pallas import tpu as pltpu

F32 = jnp.float32

B, C, G = 1024, 8, 4096
D0, D1, D2 = 512, 2048, 256
K = 1024
COM = 0.25
BT = 256


def _dot(a, b, precision=None):
    return jnp.dot(a, b, preferred_element_type=F32, precision=precision)


def _enc_kernel(x_ref, w_ref, b_ref, o_ref):
    acc = _dot(x_ref[...], w_ref[0])
    o_ref[...] = jnp.tanh(acc + b_ref[...])


def _enc(x2, enc_W, enc_b2):
    return pl.pallas_call(
        _enc_kernel,
        grid=(C, B // BT),
        in_specs=[
            pl.BlockSpec((BT, G), lambda c, b: (b, c)),
            pl.BlockSpec((1, G, D0), lambda c, b: (c, 0, 0)),
            pl.BlockSpec((1, D0), lambda c, b: (0, c)),
        ],
        out_specs=pl.BlockSpec((BT, D0), lambda c, b: (b, c)),
        out_shape=jax.ShapeDtypeStruct((B, C * D0), F32),
    )(x2, enc_W, enc_b2)


def _mlp_kernel(x_ref, w_ref, b_ref, o_ref):
    o_ref[...] = jnp.tanh(_dot(x_ref[...], w_ref[...]) + b_ref[...])


def _mlp(x, w, b, n_tiles, tile):
    m, kdim = x.shape
    n = w.shape[1]
    return pl.pallas_call(
        _mlp_kernel,
        grid=(n_tiles,),
        in_specs=[
            pl.BlockSpec((m, kdim), lambda j: (0, 0)),
            pl.BlockSpec((kdim, tile), lambda j: (0, j)),
            pl.BlockSpec((1, tile), lambda j: (0, j)),
        ],
        out_specs=pl.BlockSpec((m, tile), lambda j: (0, j)),
        out_shape=jax.ShapeDtypeStruct((m, n), F32),
    )(x, w, b)


def _vq_kernel(z1_ref, w2_ref, b2_ref, cb_ref, dw1_ref, db1_ref,
               loss_ref, qst_ref, d1_ref):
    z = jnp.tanh(_dot(z1_ref[...], w2_ref[...]) + b2_ref[...])
    cb = cb_ref[...]
    m = lax.dot_general(z, cb, (((1,), (1,)), ((), ())),
                        preferred_element_type=F32)
    distances = (jnp.sum(z * z, axis=1, keepdims=True)
                 + jnp.sum(cb * cb, axis=1)[None, :]
                 - 2.0 * m)
    mins = jnp.min(distances, axis=1, keepdims=True)
    kiota = lax.broadcasted_iota(jnp.int32, distances.shape, 1)
    idx = jnp.min(jnp.where(distances == mins, kiota, K), axis=1)
    onehot = (kiota == idx[:, None]).astype(F32)
    q = _dot(onehot, cb)
    diff = q - z
    loss = (1.0 + COM) * jnp.mean(diff * diff)
    loss_ref[...] = jnp.reshape(loss, (1, 1))
    qst = z + (q - z)
    qst_ref[...] = qst
    d1_ref[...] = jnp.tanh(_dot(qst, dw1_ref[...]) + db1_ref[...])


def _vq(z1, w2, b2, cb, dw1, db1):
    return pl.pallas_call(
        _vq_kernel,
        out_shape=(
            jax.ShapeDtypeStruct((1, 1), F32),
            jax.ShapeDtypeStruct((B, D2), F32),
            jax.ShapeDtypeStruct((B, D1), F32),
        ),
    )(z1, w2, b2, cb, dw1, db1)


def _dec_kernel(d_ref, w_ref, b_ref, o_ref):
    o_ref[...] = _dot(d_ref[...], w_ref[0]) + b_ref[...]


def _dec(d2, dec_W, dec_b2):
    return pl.pallas_call(
        _dec_kernel,
        grid=(C, B // BT),
        in_specs=[
            pl.BlockSpec((BT, D0), lambda c, b: (b, c)),
            pl.BlockSpec((1, D0, G), lambda c, b: (c, 0, 0)),
            pl.BlockSpec((1, G), lambda c, b: (0, c)),
        ],
        out_specs=pl.BlockSpec((BT, G), lambda c, b: (b, c)),
        out_shape=jax.ShapeDtypeStruct((B, C * G), F32),
    )(d2, dec_W, dec_b2)


def kernel(inputs, enc_W, enc_b, int_enc_W1, int_enc_b1, int_enc_W2,
           int_enc_b2, codebook, int_dec_W1, int_dec_b1, int_dec_W2,
           int_dec_b2, dec_W, dec_b):
    h2 = _enc(inputs.reshape(B, C * G), enc_W, enc_b.reshape(1, C * D0))
    z1 = _mlp(h2, int_enc_W1, int_enc_b1.reshape(1, D1), D1 // 512, 512)
    loss, qst, d1 = _vq(z1, int_enc_W2, int_enc_b2.reshape(1, D2),
                        codebook, int_dec_W1, int_dec_b1.reshape(1, D1))
    d2 = _mlp(d1, int_dec_W2, int_dec_b2.reshape(1, C * D0),
              (C * D0) // 512, 512)
    x_recon = _dec(d2, dec_W, dec_b.reshape(1, C * G)).reshape(B, C, G)
    return (loss[0, 0], x_recon, qst)
